# initial kernel scaffold (unmeasured)
import jax
import jax.numpy as jnp
from jax import lax
from jax.experimental import pallas as pl
from jax.experimental.pallas import tpu as pltpu

N_DEV = 8
ROW_TILE = 128


def kernel(x, w_mat):
    m, k_per = x.shape
    _, n = w_mat.shape
    m_per = m // N_DEV

    y_partial = jnp.dot(x, w_mat, preferred_element_type=jnp.float32)

    def body(y_ref, out_ref, comm_ref, stage_ref, local_sem, out_sem,
             send_sems, recv_sems, credit_sem):
        my = lax.axis_index("i")
        left = lax.rem(my + N_DEV - 1, N_DEV)
        right = lax.rem(my + 1, N_DEV)

        c0 = lax.rem(my + N_DEV - 1, N_DEV)
        cp0 = pltpu.make_async_copy(
            y_ref.at[pl.ds(c0 * m_per, m_per)], comm_ref.at[0], local_sem)
        cp0.start()

        barrier_sem = pltpu.get_barrier_semaphore()
        for nbr in (left, right):
            pl.semaphore_signal(barrier_sem, inc=1, device_id=(nbr,),
                                device_id_type=pl.DeviceIdType.MESH)
        pl.semaphore_wait(barrier_sem, 2)
        cp0.wait()

        for s in range(N_DEV - 1):
            send_slot = s % 2
            recv_slot = (s + 1) % 2
            if s >= 1:
                pl.semaphore_wait(credit_sem, 1)
            rdma = pltpu.make_async_remote_copy(
                src_ref=comm_ref.at[send_slot],
                dst_ref=comm_ref.at[recv_slot],
                send_sem=send_sems.at[send_slot],
                recv_sem=recv_sems.at[recv_slot],
                device_id=(right,),
                device_id_type=pl.DeviceIdType.MESH,
            )
            rdma.start()

            c = lax.rem(my + 2 * N_DEV - 2 - s, N_DEV)
            cp = pltpu.make_async_copy(
                y_ref.at[pl.ds(c * m_per, m_per)], stage_ref, local_sem)
            cp.start()
            cp.wait()

            rdma.wait()

            if s < N_DEV - 2:
                pl.semaphore_signal(credit_sem, inc=1, device_id=(left,),
                                    device_id_type=pl.DeviceIdType.MESH)
                for t in range(m_per // ROW_TILE):
                    sl = pl.ds(t * ROW_TILE, ROW_TILE)
                    comm_ref[recv_slot, sl, :] = (
                        comm_ref[recv_slot, sl, :] + stage_ref[sl, :])
            else:
                for t in range(m_per // ROW_TILE):
                    sl = pl.ds(t * ROW_TILE, ROW_TILE)
                    y = comm_ref[recv_slot, sl, :] + stage_ref[sl, :]
                    comm_ref[send_slot, sl, :] = y * (
                        1.0 / (1.0 + jnp.exp(-y)))
                cpout = pltpu.make_async_copy(
                    comm_ref.at[send_slot], out_ref, out_sem)
                cpout.start()
                cpout.wait()

    return pl.pallas_call(
        body,
        out_shape=jax.ShapeDtypeStruct((m_per, n), jnp.float32),
        in_specs=[pl.BlockSpec(memory_space=pltpu.MemorySpace.HBM)],
        out_specs=pl.BlockSpec(memory_space=pltpu.MemorySpace.HBM),
        scratch_shapes=[
            pltpu.VMEM((2, m_per, n), jnp.float32),
            pltpu.VMEM((m_per, n), jnp.float32),
            pltpu.SemaphoreType.DMA,
            pltpu.SemaphoreType.DMA,
            pltpu.SemaphoreType.DMA((2,)),
            pltpu.SemaphoreType.DMA((2,)),
            pltpu.SemaphoreType.REGULAR,
        ],
        compiler_params=pltpu.CompilerParams(collective_id=0),
    )(y_partial)


# baseline (device time: 1369403 ns/iter reference)
import jax
import jax.numpy as jnp
from jax import lax
from jax.experimental import pallas as pl
from jax.experimental.pallas import tpu as pltpu

N_DEV = 8
ROW_TILE = 128


def kernel(x, w_mat):
    m, k_per = x.shape
    _, n = w_mat.shape
    m_per = m // N_DEV

    y_partial = jnp.dot(x, w_mat, preferred_element_type=jnp.float32)

    def body(y_ref, out_ref, comm_ref, stage_ref, local_sem, out_sem,
             send_sems, recv_sems, credit_sem):
        my = lax.axis_index("i")
        left = lax.rem(my + N_DEV - 1, N_DEV)
        right = lax.rem(my + 1, N_DEV)

        c0 = lax.rem(my + N_DEV - 1, N_DEV)
        cp0 = pltpu.make_async_copy(
            y_ref.at[pl.ds(c0 * m_per, m_per)], comm_ref.at[0], local_sem)
        cp0.start()

        barrier_sem = pltpu.get_barrier_semaphore()
        for nbr in (left, right):
            pl.semaphore_signal(barrier_sem, inc=1, device_id=(nbr,),
                                device_id_type=pl.DeviceIdType.MESH)
        pl.semaphore_wait(barrier_sem, 2)
        cp0.wait()

        for s in range(N_DEV - 1):
            send_slot = s % 2
            recv_slot = (s + 1) % 2
            if s >= 1:
                pl.semaphore_wait(credit_sem, 1)
            rdma = pltpu.make_async_remote_copy(
                src_ref=comm_ref.at[send_slot],
                dst_ref=comm_ref.at[recv_slot],
                send_sem=send_sems.at[send_slot],
                recv_sem=recv_sems.at[recv_slot],
                device_id=(right,),
                device_id_type=pl.DeviceIdType.MESH,
            )
            rdma.start()

            c = lax.rem(my + 2 * N_DEV - 2 - s, N_DEV)
            cp = pltpu.make_async_copy(
                y_ref.at[pl.ds(c * m_per, m_per)], stage_ref, local_sem)
            cp.start()
            cp.wait()

            rdma.wait()

            if s < N_DEV - 2:
                pl.semaphore_signal(credit_sem, inc=1, device_id=(left,),
                                    device_id_type=pl.DeviceIdType.MESH)
                for t in range(m_per // ROW_TILE):
                    sl = pl.ds(t * ROW_TILE, ROW_TILE)
                    comm_ref[recv_slot, sl, :] = (
                        comm_ref[recv_slot, sl, :] + stage_ref[sl, :])
            else:
                for t in range(m_per // ROW_TILE):
                    sl = pl.ds(t * ROW_TILE, ROW_TILE)
                    y = comm_ref[recv_slot, sl, :] + stage_ref[sl, :]
                    comm_ref[send_slot, sl, :] = y * (
                        1.0 / (1.0 + jnp.exp(-y)))
                cpout = pltpu.make_async_copy(
                    comm_ref.at[send_slot], out_ref, out_sem)
                cpout.start()
                cpout.wait()

    return pl.pallas_call(
        body,
        out_shape=jax.ShapeDtypeStruct((m_per, n), jnp.float32),
        in_specs=[pl.BlockSpec(memory_space=pltpu.MemorySpace.HBM)],
        out_specs=pl.BlockSpec(memory_space=pltpu.MemorySpace.HBM),
        scratch_shapes=[
            pltpu.VMEM((2, m_per, n), jnp.float32),
            pltpu.VMEM((m_per, n), jnp.float32),
            pltpu.SemaphoreType.DMA,
            pltpu.SemaphoreType.DMA,
            pltpu.SemaphoreType.DMA((2,)),
            pltpu.SemaphoreType.DMA((2,)),
            pltpu.SemaphoreType.REGULAR,
        ],
        compiler_params=pltpu.CompilerParams(
            collective_id=0, vmem_limit_bytes=60 * 1024 * 1024),
    )(y_partial)


# device time: 741046 ns/iter; 1.8479x vs baseline; 1.8479x over previous
import jax
import jax.numpy as jnp
from jax import lax
from jax.experimental import pallas as pl
from jax.experimental.pallas import tpu as pltpu

N_DEV = 8
ROW_TILE = 128


def kernel(x, w_mat):
    m, k_per = x.shape
    _, n = w_mat.shape
    m_per = m // N_DEV
    half = n // 2

    y_partial = jnp.dot(x, w_mat, preferred_element_type=jnp.float32)

    def body(y_ref, out_ref, comm_cw, comm_ccw, stage_cw, stage_ccw,
             local_sems, out_sems, send_cw, recv_cw, send_ccw, recv_ccw,
             credit_cw, credit_ccw):
        my = lax.axis_index("i")
        left = lax.rem(my + N_DEV - 1, N_DEV)
        right = lax.rem(my + 1, N_DEV)

        cw0 = lax.rem(my + N_DEV - 1, N_DEV)
        ccw0 = lax.rem(my + 1, N_DEV)
        cp_cw0 = pltpu.make_async_copy(
            y_ref.at[pl.ds(cw0 * m_per, m_per), pl.ds(0, half)],
            comm_cw.at[0], local_sems.at[0])
        cp_ccw0 = pltpu.make_async_copy(
            y_ref.at[pl.ds(ccw0 * m_per, m_per), pl.ds(half, half)],
            comm_ccw.at[0], local_sems.at[1])
        cp_cw0.start()
        cp_ccw0.start()

        barrier_sem = pltpu.get_barrier_semaphore()
        for nbr in (left, right):
            pl.semaphore_signal(barrier_sem, inc=1, device_id=(nbr,),
                                device_id_type=pl.DeviceIdType.MESH)
        pl.semaphore_wait(barrier_sem, 2)
        cp_cw0.wait()
        cp_ccw0.wait()

        for s in range(N_DEV - 1):
            send_slot = s % 2
            recv_slot = (s + 1) % 2
            if s >= 1:
                pl.semaphore_wait(credit_cw, 1)
                pl.semaphore_wait(credit_ccw, 1)
            rdma_cw = pltpu.make_async_remote_copy(
                src_ref=comm_cw.at[send_slot],
                dst_ref=comm_cw.at[recv_slot],
                send_sem=send_cw.at[send_slot],
                recv_sem=recv_cw.at[recv_slot],
                device_id=(right,),
                device_id_type=pl.DeviceIdType.MESH,
            )
            rdma_ccw = pltpu.make_async_remote_copy(
                src_ref=comm_ccw.at[send_slot],
                dst_ref=comm_ccw.at[recv_slot],
                send_sem=send_ccw.at[send_slot],
                recv_sem=recv_ccw.at[recv_slot],
                device_id=(left,),
                device_id_type=pl.DeviceIdType.MESH,
            )
            rdma_cw.start()
            rdma_ccw.start()

            c_cw = lax.rem(my + 2 * N_DEV - 2 - s, N_DEV)
            c_ccw = lax.rem(my + 2 + s, N_DEV)
            cp_cw = pltpu.make_async_copy(
                y_ref.at[pl.ds(c_cw * m_per, m_per), pl.ds(0, half)],
                stage_cw, local_sems.at[0])
            cp_ccw = pltpu.make_async_copy(
                y_ref.at[pl.ds(c_ccw * m_per, m_per), pl.ds(half, half)],
                stage_ccw, local_sems.at[1])
            cp_cw.start()
            cp_ccw.start()
            cp_cw.wait()
            cp_ccw.wait()

            rdma_cw.wait()
            rdma_ccw.wait()

            if s < N_DEV - 2:
                pl.semaphore_signal(credit_cw, inc=1, device_id=(left,),
                                    device_id_type=pl.DeviceIdType.MESH)
                pl.semaphore_signal(credit_ccw, inc=1, device_id=(right,),
                                    device_id_type=pl.DeviceIdType.MESH)
                for t in range(m_per // ROW_TILE):
                    sl = pl.ds(t * ROW_TILE, ROW_TILE)
                    comm_cw[recv_slot, sl, :] = (
                        comm_cw[recv_slot, sl, :] + stage_cw[sl, :])
                    comm_ccw[recv_slot, sl, :] = (
                        comm_ccw[recv_slot, sl, :] + stage_ccw[sl, :])
            else:
                for t in range(m_per // ROW_TILE):
                    sl = pl.ds(t * ROW_TILE, ROW_TILE)
                    ycw = comm_cw[recv_slot, sl, :] + stage_cw[sl, :]
                    comm_cw[send_slot, sl, :] = ycw * (
                        1.0 / (1.0 + jnp.exp(-ycw)))
                    yccw = comm_ccw[recv_slot, sl, :] + stage_ccw[sl, :]
                    comm_ccw[send_slot, sl, :] = yccw * (
                        1.0 / (1.0 + jnp.exp(-yccw)))
                cp_out_cw = pltpu.make_async_copy(
                    comm_cw.at[send_slot],
                    out_ref.at[:, pl.ds(0, half)], out_sems.at[0])
                cp_out_ccw = pltpu.make_async_copy(
                    comm_ccw.at[send_slot],
                    out_ref.at[:, pl.ds(half, half)], out_sems.at[1])
                cp_out_cw.start()
                cp_out_ccw.start()
                cp_out_cw.wait()
                cp_out_ccw.wait()

    return pl.pallas_call(
        body,
        out_shape=jax.ShapeDtypeStruct((m_per, n), jnp.float32),
        in_specs=[pl.BlockSpec(memory_space=pltpu.MemorySpace.HBM)],
        out_specs=pl.BlockSpec(memory_space=pltpu.MemorySpace.HBM),
        scratch_shapes=[
            pltpu.VMEM((2, m_per, half), jnp.float32),
            pltpu.VMEM((2, m_per, half), jnp.float32),
            pltpu.VMEM((m_per, half), jnp.float32),
            pltpu.VMEM((m_per, half), jnp.float32),
            pltpu.SemaphoreType.DMA((2,)),
            pltpu.SemaphoreType.DMA((2,)),
            pltpu.SemaphoreType.DMA((2,)),
            pltpu.SemaphoreType.DMA((2,)),
            pltpu.SemaphoreType.DMA((2,)),
            pltpu.SemaphoreType.DMA((2,)),
            pltpu.SemaphoreType.REGULAR,
            pltpu.SemaphoreType.REGULAR,
        ],
        compiler_params=pltpu.CompilerParams(
            collective_id=0, vmem_limit_bytes=60 * 1024 * 1024),
    )(y_partial)


# device time: 692958 ns/iter; 1.9762x vs baseline; 1.0694x over previous
import jax
import jax.numpy as jnp
from jax import lax
from jax.experimental import pallas as pl
from jax.experimental.pallas import tpu as pltpu

N_DEV = 8
ROW_TILE = 128
W_TILE = 1024


def kernel(x, w_mat):
    m, k_per = x.shape
    _, n = w_mat.shape
    m_per = m // N_DEV
    half = n // 2
    nq = half // W_TILE

    def body(x_ref, w_ref, out_ref, comm_cw, comm_ccw, p_cw, p_ccw,
             x_stage, w_stage, x_sems, w_sems, out_sems,
             send_cw, recv_cw, send_ccw, recv_ccw, credit_cw, credit_ccw):
        my = lax.axis_index("i")
        left = lax.rem(my + N_DEV - 1, N_DEV)
        right = lax.rem(my + 1, N_DEV)

        def start_x(c, slot):
            cp = pltpu.make_async_copy(
                x_ref.at[pl.ds(c * m_per, m_per)], x_stage.at[slot],
                x_sems.at[slot])
            cp.start()
            return cp

        def compute_half(x_slot, w_col0, dst):
            cps = [None, None]
            cps[0] = pltpu.make_async_copy(
                w_ref.at[:, pl.ds(w_col0, W_TILE)], w_stage.at[0],
                w_sems.at[0])
            cps[0].start()
            xs = x_stage[x_slot]
            for q in range(nq):
                if q + 1 < nq:
                    nxt = (q + 1) % 2
                    cps[nxt] = pltpu.make_async_copy(
                        w_ref.at[:, pl.ds(w_col0 + (q + 1) * W_TILE, W_TILE)],
                        w_stage.at[nxt], w_sems.at[nxt])
                    cps[nxt].start()
                cps[q % 2].wait()
                dst[:, pl.ds(q * W_TILE, W_TILE)] = jnp.dot(
                    xs, w_stage[q % 2], preferred_element_type=jnp.float32)

        cw0 = lax.rem(my + N_DEV - 1, N_DEV)
        ccw0 = lax.rem(my + 1, N_DEV)
        cpx0 = start_x(cw0, 0)
        cpx1 = start_x(ccw0, 1)

        barrier_sem = pltpu.get_barrier_semaphore()
        for nbr in (left, right):
            pl.semaphore_signal(barrier_sem, inc=1, device_id=(nbr,),
                                device_id_type=pl.DeviceIdType.MESH)
        pl.semaphore_wait(barrier_sem, 2)

        cpx0.wait()
        compute_half(0, 0, comm_cw.at[0])
        cpx1.wait()
        compute_half(1, half, comm_ccw.at[0])

        for s in range(N_DEV - 1):
            send_slot = s % 2
            recv_slot = (s + 1) % 2
            if s >= 1:
                pl.semaphore_wait(credit_cw, 1)
                pl.semaphore_wait(credit_ccw, 1)
            rdma_cw = pltpu.make_async_remote_copy(
                src_ref=comm_cw.at[send_slot],
                dst_ref=comm_cw.at[recv_slot],
                send_sem=send_cw.at[send_slot],
                recv_sem=recv_cw.at[recv_slot],
                device_id=(right,),
                device_id_type=pl.DeviceIdType.MESH,
            )
            rdma_ccw = pltpu.make_async_remote_copy(
                src_ref=comm_ccw.at[send_slot],
                dst_ref=comm_ccw.at[recv_slot],
                send_sem=send_ccw.at[send_slot],
                recv_sem=recv_ccw.at[recv_slot],
                device_id=(left,),
                device_id_type=pl.DeviceIdType.MESH,
            )
            rdma_cw.start()
            rdma_ccw.start()

            c_cw = lax.rem(my + 2 * N_DEV - 2 - s, N_DEV)
            c_ccw = lax.rem(my + 2 + s, N_DEV)
            cpx_cw = start_x(c_cw, 0)
            cpx_ccw = start_x(c_ccw, 1)
            cpx_cw.wait()
            compute_half(0, 0, p_cw)
            cpx_ccw.wait()
            compute_half(1, half, p_ccw)

            rdma_cw.wait()
            rdma_ccw.wait()

            if s < N_DEV - 2:
                pl.semaphore_signal(credit_cw, inc=1, device_id=(left,),
                                    device_id_type=pl.DeviceIdType.MESH)
                pl.semaphore_signal(credit_ccw, inc=1, device_id=(right,),
                                    device_id_type=pl.DeviceIdType.MESH)
                for t in range(m_per // ROW_TILE):
                    sl = pl.ds(t * ROW_TILE, ROW_TILE)
                    comm_cw[recv_slot, sl, :] = (
                        comm_cw[recv_slot, sl, :] + p_cw[sl, :])
                    comm_ccw[recv_slot, sl, :] = (
                        comm_ccw[recv_slot, sl, :] + p_ccw[sl, :])
            else:
                for t in range(m_per // ROW_TILE):
                    sl = pl.ds(t * ROW_TILE, ROW_TILE)
                    ycw = comm_cw[recv_slot, sl, :] + p_cw[sl, :]
                    comm_cw[send_slot, sl, :] = ycw * (
                        1.0 / (1.0 + jnp.exp(-ycw)))
                    yccw = comm_ccw[recv_slot, sl, :] + p_ccw[sl, :]
                    comm_ccw[send_slot, sl, :] = yccw * (
                        1.0 / (1.0 + jnp.exp(-yccw)))
                cp_out_cw = pltpu.make_async_copy(
                    comm_cw.at[send_slot],
                    out_ref.at[:, pl.ds(0, half)], out_sems.at[0])
                cp_out_ccw = pltpu.make_async_copy(
                    comm_ccw.at[send_slot],
                    out_ref.at[:, pl.ds(half, half)], out_sems.at[1])
                cp_out_cw.start()
                cp_out_ccw.start()
                cp_out_cw.wait()
                cp_out_ccw.wait()

    return pl.pallas_call(
        body,
        out_shape=jax.ShapeDtypeStruct((m_per, n), jnp.float32),
        in_specs=[
            pl.BlockSpec(memory_space=pltpu.MemorySpace.HBM),
            pl.BlockSpec(memory_space=pltpu.MemorySpace.HBM),
        ],
        out_specs=pl.BlockSpec(memory_space=pltpu.MemorySpace.HBM),
        scratch_shapes=[
            pltpu.VMEM((2, m_per, half), jnp.float32),
            pltpu.VMEM((2, m_per, half), jnp.float32),
            pltpu.VMEM((m_per, half), jnp.float32),
            pltpu.VMEM((m_per, half), jnp.float32),
            pltpu.VMEM((2, m_per, k_per), jnp.float32),
            pltpu.VMEM((2, k_per, W_TILE), jnp.float32),
            pltpu.SemaphoreType.DMA((2,)),
            pltpu.SemaphoreType.DMA((2,)),
            pltpu.SemaphoreType.DMA((2,)),
            pltpu.SemaphoreType.DMA((2,)),
            pltpu.SemaphoreType.DMA((2,)),
            pltpu.SemaphoreType.DMA((2,)),
            pltpu.SemaphoreType.DMA((2,)),
            pltpu.SemaphoreType.REGULAR,
            pltpu.SemaphoreType.REGULAR,
        ],
        compiler_params=pltpu.CompilerParams(
            collective_id=0, vmem_limit_bytes=64 * 1024 * 1024),
    )(x, w_mat)


# device time: 661948 ns/iter; 2.0687x vs baseline; 1.0468x over previous
import jax
import jax.numpy as jnp
from jax import lax
from jax.experimental import pallas as pl
from jax.experimental.pallas import tpu as pltpu

N_DEV = 8
ROW_TILE = 128
W_TILE = 1024
N_SUB = 2


def kernel(x, w_mat):
    m, k_per = x.shape
    _, n = w_mat.shape
    m_per = m // N_DEV
    half = n // 2
    nq = half // W_TILE
    sbw = half // N_SUB

    def body(x_ref, w_ref, out_ref, comm_cw, comm_ccw, p_cw, p_ccw,
             x_stage, w_stage, x_sems, w_sems, out_sems,
             send_cw, recv_cw, send_ccw, recv_ccw, credit_cw, credit_ccw):
        my = lax.axis_index("i")
        left = lax.rem(my + N_DEV - 1, N_DEV)
        right = lax.rem(my + 1, N_DEV)

        def start_x(c, slot):
            cp = pltpu.make_async_copy(
                x_ref.at[pl.ds(c * m_per, m_per)], x_stage.at[slot],
                x_sems.at[slot])
            cp.start()
            return cp

        def compute_half(x_slot, w_col0, dst):
            cps = [None, None]
            cps[0] = pltpu.make_async_copy(
                w_ref.at[:, pl.ds(w_col0, W_TILE)], w_stage.at[0],
                w_sems.at[0])
            cps[0].start()
            xs = x_stage[x_slot]
            for q in range(nq):
                if q + 1 < nq:
                    nxt = (q + 1) % 2
                    cps[nxt] = pltpu.make_async_copy(
                        w_ref.at[:, pl.ds(w_col0 + (q + 1) * W_TILE, W_TILE)],
                        w_stage.at[nxt], w_sems.at[nxt])
                    cps[nxt].start()
                cps[q % 2].wait()
                dst[:, pl.ds(q * W_TILE, W_TILE)] = jnp.dot(
                    xs, w_stage[q % 2], preferred_element_type=jnp.float32)

        def mk(is_cw, s, b):
            send_slot = s % 2
            recv_slot = (s + 1) % 2
            comm = comm_cw if is_cw else comm_ccw
            ssem = send_cw if is_cw else send_ccw
            rsem = recv_cw if is_cw else recv_ccw
            tgt = right if is_cw else left
            cs = pl.ds(b * sbw, sbw)
            return pltpu.make_async_remote_copy(
                src_ref=comm.at[send_slot, :, cs],
                dst_ref=comm.at[recv_slot, :, cs],
                send_sem=ssem.at[send_slot, b],
                recv_sem=rsem.at[recv_slot, b],
                device_id=(tgt,),
                device_id_type=pl.DeviceIdType.MESH,
            )

        cw0 = lax.rem(my + N_DEV - 1, N_DEV)
        ccw0 = lax.rem(my + 1, N_DEV)
        cpx0 = start_x(cw0, 0)
        cpx1 = start_x(ccw0, 1)

        barrier_sem = pltpu.get_barrier_semaphore()
        for nbr in (left, right):
            pl.semaphore_signal(barrier_sem, inc=1, device_id=(nbr,),
                                device_id_type=pl.DeviceIdType.MESH)
        pl.semaphore_wait(barrier_sem, 2)

        cpx0.wait()
        compute_half(0, 0, comm_cw.at[0])
        d_cw = [mk(True, 0, 0), mk(True, 0, 1)]
        d_cw[0].start()
        d_cw[1].start()
        cpx1.wait()
        compute_half(1, half, comm_ccw.at[0])
        d_ccw = [mk(False, 0, 0), mk(False, 0, 1)]
        d_ccw[0].start()
        d_ccw[1].start()

        out_cps = []
        for s in range(N_DEV - 1):
            send_slot = s % 2
            recv_slot = (s + 1) % 2
            last = s == N_DEV - 2

            c_cw = lax.rem(my + 2 * N_DEV - 2 - s, N_DEV)
            c_ccw = lax.rem(my + 2 + s, N_DEV)
            cpx_cw = start_x(c_cw, 0)
            cpx_ccw = start_x(c_ccw, 1)
            cpx_cw.wait()
            compute_half(0, 0, p_cw)
            cpx_ccw.wait()
            compute_half(1, half, p_ccw)

            nd_cw = [None, None]
            nd_ccw = [None, None]
            for b in range(N_SUB):
                cs = pl.ds(b * sbw, sbw)
                for is_cw in (True, False):
                    d = d_cw if is_cw else d_ccw
                    comm = comm_cw if is_cw else comm_ccw
                    p = p_cw if is_cw else p_ccw
                    credit = credit_cw if is_cw else credit_ccw
                    upstream = left if is_cw else right
                    nd = nd_cw if is_cw else nd_ccw
                    out_col0 = b * sbw if is_cw else half + b * sbw

                    d[b].wait_recv()
                    if not last:
                        for t in range(m_per // ROW_TILE):
                            sl = pl.ds(t * ROW_TILE, ROW_TILE)
                            comm[recv_slot, sl, cs] = (
                                comm[recv_slot, sl, cs] + p[sl, cs])
                    else:
                        for t in range(m_per // ROW_TILE):
                            sl = pl.ds(t * ROW_TILE, ROW_TILE)
                            y = comm[recv_slot, sl, cs] + p[sl, cs]
                            comm[send_slot, sl, cs] = y * (
                                1.0 / (1.0 + jnp.exp(-y)))
                        cp_out = pltpu.make_async_copy(
                            comm.at[send_slot, :, cs],
                            out_ref.at[:, pl.ds(out_col0, sbw)],
                            out_sems.at[0 if is_cw else 1, b])
                        cp_out.start()
                        out_cps.append(cp_out)

                    d[b].wait_send()
                    if s < N_DEV - 2:
                        pl.semaphore_signal(
                            credit, inc=1, device_id=(upstream,),
                            device_id_type=pl.DeviceIdType.MESH)
                    if not last:
                        pl.semaphore_wait(credit, 1)
                        nd[b] = mk(is_cw, s + 1, b)
                        nd[b].start()
            d_cw, d_ccw = nd_cw, nd_ccw

        for cp_out in out_cps:
            cp_out.wait()

    return pl.pallas_call(
        body,
        out_shape=jax.ShapeDtypeStruct((m_per, n), jnp.float32),
        in_specs=[
            pl.BlockSpec(memory_space=pltpu.MemorySpace.HBM),
            pl.BlockSpec(memory_space=pltpu.MemorySpace.HBM),
        ],
        out_specs=pl.BlockSpec(memory_space=pltpu.MemorySpace.HBM),
        scratch_shapes=[
            pltpu.VMEM((2, m_per, half), jnp.float32),
            pltpu.VMEM((2, m_per, half), jnp.float32),
            pltpu.VMEM((m_per, half), jnp.float32),
            pltpu.VMEM((m_per, half), jnp.float32),
            pltpu.VMEM((2, m_per, k_per), jnp.float32),
            pltpu.VMEM((2, k_per, W_TILE), jnp.float32),
            pltpu.SemaphoreType.DMA((2,)),
            pltpu.SemaphoreType.DMA((2,)),
            pltpu.SemaphoreType.DMA((2, N_SUB)),
            pltpu.SemaphoreType.DMA((2, N_SUB)),
            pltpu.SemaphoreType.DMA((2, N_SUB)),
            pltpu.SemaphoreType.DMA((2, N_SUB)),
            pltpu.SemaphoreType.DMA((2, N_SUB)),
            pltpu.SemaphoreType.REGULAR,
            pltpu.SemaphoreType.REGULAR,
        ],
        compiler_params=pltpu.CompilerParams(
            collective_id=0, vmem_limit_bytes=64 * 1024 * 1024),
    )(x, w_mat)


# device time: 657898 ns/iter; 2.0815x vs baseline; 1.0062x over previous
import jax
import jax.numpy as jnp
from jax import lax
from jax.experimental import pallas as pl
from jax.experimental.pallas import tpu as pltpu

N_DEV = 8
ROW_TILE = 256
W_TILE = 1024
N_SUB = 4


def kernel(x, w_mat):
    m, k_per = x.shape
    _, n = w_mat.shape
    m_per = m // N_DEV
    half = n // 2
    nq = half // W_TILE
    sbw = half // N_SUB

    def body(x_ref, w_ref, out_ref, comm_cw, comm_ccw, p_cw, p_ccw,
             x_stage, w_stage, x_sems, w_sems, out_sems,
             send_cw, recv_cw, send_ccw, recv_ccw, credit_cw, credit_ccw):
        my = lax.axis_index("i")
        left = lax.rem(my + N_DEV - 1, N_DEV)
        right = lax.rem(my + 1, N_DEV)

        def start_x(c, slot):
            cp = pltpu.make_async_copy(
                x_ref.at[pl.ds(c * m_per, m_per)], x_stage.at[slot],
                x_sems.at[slot])
            cp.start()
            return cp

        def compute_half(x_slot, w_col0, dst):
            cps = [None, None]
            cps[0] = pltpu.make_async_copy(
                w_ref.at[:, pl.ds(w_col0, W_TILE)], w_stage.at[0],
                w_sems.at[0])
            cps[0].start()
            xs = x_stage[x_slot]
            for q in range(nq):
                if q + 1 < nq:
                    nxt = (q + 1) % 2
                    cps[nxt] = pltpu.make_async_copy(
                        w_ref.at[:, pl.ds(w_col0 + (q + 1) * W_TILE, W_TILE)],
                        w_stage.at[nxt], w_sems.at[nxt])
                    cps[nxt].start()
                cps[q % 2].wait()
                dst[:, pl.ds(q * W_TILE, W_TILE)] = jnp.dot(
                    xs, w_stage[q % 2], preferred_element_type=jnp.float32)

        def mk(is_cw, s, b):
            send_slot = s % 2
            recv_slot = (s + 1) % 2
            comm = comm_cw if is_cw else comm_ccw
            ssem = send_cw if is_cw else send_ccw
            rsem = recv_cw if is_cw else recv_ccw
            tgt = right if is_cw else left
            cs = pl.ds(b * sbw, sbw)
            return pltpu.make_async_remote_copy(
                src_ref=comm.at[send_slot, :, cs],
                dst_ref=comm.at[recv_slot, :, cs],
                send_sem=ssem.at[send_slot, b],
                recv_sem=rsem.at[recv_slot, b],
                device_id=(tgt,),
                device_id_type=pl.DeviceIdType.MESH,
            )

        cw0 = lax.rem(my + N_DEV - 1, N_DEV)
        ccw0 = lax.rem(my + 1, N_DEV)
        cpx0 = start_x(cw0, 0)
        cpx1 = start_x(ccw0, 1)

        barrier_sem = pltpu.get_barrier_semaphore()
        for nbr in (left, right):
            pl.semaphore_signal(barrier_sem, inc=1, device_id=(nbr,),
                                device_id_type=pl.DeviceIdType.MESH)
        pl.semaphore_wait(barrier_sem, 2)

        def preload_dir(is_cw, x_slot, w_col0, comm):
            ds = [None] * N_SUB
            cps = [None, None]
            cps[0] = pltpu.make_async_copy(
                w_ref.at[:, pl.ds(w_col0, W_TILE)], w_stage.at[0],
                w_sems.at[0])
            cps[0].start()
            xs = x_stage[x_slot]
            for q in range(nq):
                if q + 1 < nq:
                    nxt = (q + 1) % 2
                    cps[nxt] = pltpu.make_async_copy(
                        w_ref.at[:, pl.ds(w_col0 + (q + 1) * W_TILE, W_TILE)],
                        w_stage.at[nxt], w_sems.at[nxt])
                    cps[nxt].start()
                cps[q % 2].wait()
                comm[0, :, pl.ds(q * W_TILE, W_TILE)] = jnp.dot(
                    xs, w_stage[q % 2], preferred_element_type=jnp.float32)
                ds[q] = mk(is_cw, 0, q)
                ds[q].start()
            return ds

        cpx0.wait()
        d_cw = preload_dir(True, 0, 0, comm_cw)
        cpx1.wait()
        d_ccw = preload_dir(False, 1, half, comm_ccw)

        out_cps = []
        for s in range(N_DEV - 1):
            send_slot = s % 2
            recv_slot = (s + 1) % 2
            last = s == N_DEV - 2

            c_cw = lax.rem(my + 2 * N_DEV - 2 - s, N_DEV)
            c_ccw = lax.rem(my + 2 + s, N_DEV)
            cpx_cw = start_x(c_cw, 0)
            cpx_ccw = start_x(c_ccw, 1)
            cpx_cw.wait()
            compute_half(0, 0, p_cw)
            cpx_ccw.wait()
            compute_half(1, half, p_ccw)

            nd_cw = [None] * N_SUB
            nd_ccw = [None] * N_SUB
            for b in range(N_SUB):
                cs = pl.ds(b * sbw, sbw)
                for is_cw in (True, False):
                    d = d_cw if is_cw else d_ccw
                    comm = comm_cw if is_cw else comm_ccw
                    p = p_cw if is_cw else p_ccw
                    credit = credit_cw if is_cw else credit_ccw
                    upstream = left if is_cw else right
                    nd = nd_cw if is_cw else nd_ccw
                    out_col0 = b * sbw if is_cw else half + b * sbw

                    d[b].wait_recv()
                    if not last:
                        for t in range(m_per // ROW_TILE):
                            sl = pl.ds(t * ROW_TILE, ROW_TILE)
                            comm[recv_slot, sl, cs] = (
                                comm[recv_slot, sl, cs] + p[sl, cs])
                    else:
                        for t in range(m_per // ROW_TILE):
                            sl = pl.ds(t * ROW_TILE, ROW_TILE)
                            y = comm[recv_slot, sl, cs] + p[sl, cs]
                            comm[send_slot, sl, cs] = y * (
                                1.0 / (1.0 + jnp.exp(-y)))
                        cp_out = pltpu.make_async_copy(
                            comm.at[send_slot, :, cs],
                            out_ref.at[:, pl.ds(out_col0, sbw)],
                            out_sems.at[0 if is_cw else 1, b])
                        cp_out.start()
                        out_cps.append(cp_out)

                    d[b].wait_send()
                    if s < N_DEV - 2:
                        pl.semaphore_signal(
                            credit, inc=1, device_id=(upstream,),
                            device_id_type=pl.DeviceIdType.MESH)
                    if not last:
                        pl.semaphore_wait(credit, 1)
                        nd[b] = mk(is_cw, s + 1, b)
                        nd[b].start()
            d_cw, d_ccw = nd_cw, nd_ccw

        for cp_out in out_cps:
            cp_out.wait()

    return pl.pallas_call(
        body,
        out_shape=jax.ShapeDtypeStruct((m_per, n), jnp.float32),
        in_specs=[
            pl.BlockSpec(memory_space=pltpu.MemorySpace.HBM),
            pl.BlockSpec(memory_space=pltpu.MemorySpace.HBM),
        ],
        out_specs=pl.BlockSpec(memory_space=pltpu.MemorySpace.HBM),
        scratch_shapes=[
            pltpu.VMEM((2, m_per, half), jnp.float32),
            pltpu.VMEM((2, m_per, half), jnp.float32),
            pltpu.VMEM((m_per, half), jnp.float32),
            pltpu.VMEM((m_per, half), jnp.float32),
            pltpu.VMEM((2, m_per, k_per), jnp.float32),
            pltpu.VMEM((2, k_per, W_TILE), jnp.float32),
            pltpu.SemaphoreType.DMA((2,)),
            pltpu.SemaphoreType.DMA((2,)),
            pltpu.SemaphoreType.DMA((2, N_SUB)),
            pltpu.SemaphoreType.DMA((2, N_SUB)),
            pltpu.SemaphoreType.DMA((2, N_SUB)),
            pltpu.SemaphoreType.DMA((2, N_SUB)),
            pltpu.SemaphoreType.DMA((2, N_SUB)),
            pltpu.SemaphoreType.REGULAR,
            pltpu.SemaphoreType.REGULAR,
        ],
        compiler_params=pltpu.CompilerParams(
            collective_id=0, vmem_limit_bytes=64 * 1024 * 1024),
    )(x, w_mat)
